# baseline (speedup 1.0000x reference)
import functools

import jax
import jax.numpy as jnp
from jax.experimental import pallas as pl
from jax.experimental.pallas import tpu as pltpu

_EPS = 1e-5


def _conv3(y, acat, T):
    R, C = y.shape
    t = jax.lax.broadcasted_iota(jnp.int32, (R, 1), 0) % T
    z = jnp.zeros((1, C), jnp.bfloat16)
    zero = jnp.zeros((), jnp.bfloat16)
    yd = jnp.where(t == 0, zero, jnp.concatenate([z, y[:-1]], axis=0))
    yu = jnp.where(t == T - 1, zero, jnp.concatenate([y[1:], z], axis=0))
    ycat = jnp.concatenate([yd, y, yu], axis=1)
    return jnp.dot(ycat, acat, preferred_element_type=jnp.float32)


def _acc_stats(st_ref, h):
    C = h.shape[1]
    h3 = h.reshape(-1, 8, C)
    blk = jnp.concatenate([jnp.sum(h3, axis=0), jnp.sum(h3 * h3, axis=0)], axis=0)

    @pl.when(pl.program_id(0) == 0)
    def _():
        st_ref[...] = jnp.zeros_like(st_ref)

    st_ref[...] += blk


def _k1(x_ref, acat_ref, b0_ref, h0_ref, st_ref, *, T):
    xf = x_ref[...].astype(jnp.bfloat16)
    h0 = _conv3(xf, acat_ref[...], T) + b0_ref[...]
    h0_ref[...] = h0.astype(jnp.bfloat16)
    _acc_stats(st_ref, h0)


def _affine_rows(st, g, be, m):
    mean = jnp.sum(st[0:8], axis=0, keepdims=True) / m
    var = jnp.sum(st[8:16], axis=0, keepdims=True) / m - mean * mean
    inv = g * jax.lax.rsqrt(var + _EPS)
    return inv, be - mean * inv


def _k2(h0_ref, st0_ref, g0_ref, be0_ref, acat_ref, b1_ref, h1_ref, st_ref,
        *, T, M):
    sc0, sh0 = _affine_rows(st0_ref[...], g0_ref[...], be0_ref[...], M)
    y0 = jnp.maximum(h0_ref[...].astype(jnp.float32) * sc0 + sh0,
                     0.0).astype(jnp.bfloat16)
    h1 = _conv3(y0, acat_ref[...], T) + b1_ref[...]
    half = h1.shape[0] // 2
    h1w = jnp.concatenate([h1[:half], h1[half:]], axis=1)
    h1_ref[...] = h1w.astype(jnp.bfloat16)
    _acc_stats(st_ref, h1w)


def _combine_taps3(s3, Bnh, T):
    v = s3.reshape(Bnh, T, 3)
    z = jnp.zeros((Bnh, 1, 1), jnp.bfloat16)
    c0 = jnp.concatenate([z, v[:, :-1, 0:1]], axis=1)
    c2 = jnp.concatenate([v[:, 1:, 2:3], z], axis=1)
    return (c0 + v[:, :, 1:2] + c2)[:, :, 0].astype(jnp.float32)


def _k3(h1_ref, st1_ref, g1_ref, be1_ref, a2bd_ref, b2_ref, out_ref, *, T, M):
    O1 = g1_ref.shape[1]
    Bnh = out_ref.shape[0] // 2
    st = st1_ref[...]
    sc1, sh1 = _affine_rows(st[:, :O1] + st[:, O1:], g1_ref[...], be1_ref[...], M)
    pack = lambda v: jnp.concatenate([v, v], axis=1)
    y1 = jnp.maximum(h1_ref[...].astype(jnp.float32) * pack(sc1) + pack(sh1), 0.0)
    s = jnp.dot(y1.astype(jnp.bfloat16), a2bd_ref[...],
                preferred_element_type=jnp.float32).astype(jnp.bfloat16)
    ct = _combine_taps3(s[:, 0:3], Bnh, T)
    cb = _combine_taps3(s[:, O1:O1 + 3], Bnh, T)
    out_ref[...] = jnp.concatenate([ct, cb], axis=0) + b2_ref[...]


def kernel(x, W0, b0, g0, be0, W1, b1, g1, be1, W2, b2):
    N, T, H = x.shape
    O0 = W0.shape[0]
    O1 = W1.shape[0]
    M = N * T
    Bn = next((b for b in (400, 200, 80, 40, 16, 8) if N % b == 0), N)
    nb = N // Bn
    R = Bn * T

    acat0 = jnp.concatenate([jnp.transpose(W0[:, :, k]) for k in range(3)],
                            axis=0).astype(jnp.bfloat16)
    acat1 = jnp.concatenate([jnp.transpose(W1[:, :, k]) for k in range(3)],
                            axis=0).astype(jnp.bfloat16)
    a2bd = jnp.zeros((2 * O1, 2 * O1), jnp.float32)
    a2bd = a2bd.at[:O1, 0:3].set(W2[0]).at[O1:, O1:O1 + 3].set(W2[0])
    a2bd = a2bd.astype(jnp.bfloat16)
    b2row = jnp.broadcast_to(b2.reshape(1, 1), (1, T)).astype(jnp.float32)

    seq = pltpu.CompilerParams(dimension_semantics=("arbitrary",))
    wspec = lambda shape: pl.BlockSpec(shape, lambda i: (0, 0))

    h0, st0 = pl.pallas_call(
        functools.partial(_k1, T=T),
        grid=(nb,),
        in_specs=[
            pl.BlockSpec((R, H), lambda i: (i, 0)),
            wspec((3 * H, O0)),
            wspec((1, O0)),
        ],
        out_specs=(
            pl.BlockSpec((R, O0), lambda i: (i, 0)),
            wspec((16, O0)),
        ),
        out_shape=(
            jax.ShapeDtypeStruct((M, O0), jnp.bfloat16),
            jax.ShapeDtypeStruct((16, O0), jnp.float32),
        ),
        compiler_params=seq,
    )(x.reshape(M, H), acat0, b0.reshape(1, -1))

    h1, st1 = pl.pallas_call(
        functools.partial(_k2, T=T, M=M),
        grid=(nb,),
        in_specs=[
            pl.BlockSpec((R, O0), lambda i: (i, 0)),
            wspec((16, O0)), wspec((1, O0)), wspec((1, O0)),
            wspec((3 * O0, O1)),
            wspec((1, O1)),
        ],
        out_specs=(
            pl.BlockSpec((R // 2, 2 * O1), lambda i: (i, 0)),
            wspec((16, 2 * O1)),
        ),
        out_shape=(
            jax.ShapeDtypeStruct((M // 2, 2 * O1), jnp.bfloat16),
            jax.ShapeDtypeStruct((16, 2 * O1), jnp.float32),
        ),
        compiler_params=seq,
    )(h0, st0, g0.reshape(1, -1), be0.reshape(1, -1), acat1, b1.reshape(1, -1))

    out2d = pl.pallas_call(
        functools.partial(_k3, T=T, M=M),
        grid=(nb,),
        in_specs=[
            pl.BlockSpec((R // 2, 2 * O1), lambda i: (i, 0)),
            wspec((16, 2 * O1)), wspec((1, O1)), wspec((1, O1)),
            wspec((2 * O1, 2 * O1)),
            wspec((1, T)),
        ],
        out_specs=pl.BlockSpec((Bn, T), lambda i: (i, 0)),
        out_shape=jax.ShapeDtypeStruct((N, T), jnp.float32),
        compiler_params=seq,
    )(h1, st1, g1.reshape(1, -1), be1.reshape(1, -1), a2bd, b2row)

    return out2d[:, :, None]


# speedup vs baseline: 1.0143x; 1.0143x over previous
import functools

import jax
import jax.numpy as jnp
from jax.experimental import pallas as pl
from jax.experimental.pallas import tpu as pltpu

_EPS = 1e-5


def _conv3(y, acat, T):
    R, C = y.shape
    t = jax.lax.broadcasted_iota(jnp.int32, (R, 1), 0) % T
    z = jnp.zeros((1, C), jnp.bfloat16)
    zero = jnp.zeros((), jnp.bfloat16)
    yd = jnp.where(t == 0, zero, jnp.concatenate([z, y[:-1]], axis=0))
    yu = jnp.where(t == T - 1, zero, jnp.concatenate([y[1:], z], axis=0))
    ycat = jnp.concatenate([yd, y, yu], axis=1)
    return jnp.dot(ycat, acat, preferred_element_type=jnp.float32)


def _acc_stats(st_ref, h):
    C = h.shape[1]
    h3 = h.reshape(-1, 8, C)
    blk = jnp.concatenate([jnp.sum(h3, axis=0), jnp.sum(h3 * h3, axis=0)], axis=0)

    @pl.when(pl.program_id(0) == 0)
    def _():
        st_ref[...] = jnp.zeros_like(st_ref)

    st_ref[...] += blk


def _k1(x_ref, acat_ref, b0_ref, h0_ref, st_ref, *, T):
    xf = x_ref[...].astype(jnp.bfloat16)
    h0 = _conv3(xf, acat_ref[...], T) + b0_ref[...]
    h0_ref[...] = h0.astype(jnp.bfloat16)
    _acc_stats(st_ref, h0)


def _affine_rows(st, g, be, m):
    mean = jnp.sum(st[0:8], axis=0, keepdims=True) / m
    var = jnp.sum(st[8:16], axis=0, keepdims=True) / m - mean * mean
    inv = g * jax.lax.rsqrt(var + _EPS)
    return inv, be - mean * inv


def _k2(h0_ref, st0_ref, g0_ref, be0_ref, acat_ref, b1_ref, h1_ref, st_ref,
        *, T, M):
    sc0, sh0 = _affine_rows(st0_ref[...], g0_ref[...], be0_ref[...], M)
    y0 = jnp.maximum(h0_ref[...] * sc0.astype(jnp.bfloat16)
                     + sh0.astype(jnp.bfloat16), jnp.bfloat16(0))
    h1 = _conv3(y0, acat_ref[...], T) + b1_ref[...]
    half = h1.shape[0] // 2
    h1w = jnp.concatenate([h1[:half], h1[half:]], axis=1)
    h1_ref[...] = h1w.astype(jnp.bfloat16)
    _acc_stats(st_ref, h1w)


def _combine_taps3(s3, Bnh, T):
    v = s3.reshape(Bnh, T, 3)
    z = jnp.zeros((Bnh, 1, 1), jnp.bfloat16)
    c0 = jnp.concatenate([z, v[:, :-1, 0:1]], axis=1)
    c2 = jnp.concatenate([v[:, 1:, 2:3], z], axis=1)
    return (c0 + v[:, :, 1:2] + c2)[:, :, 0].astype(jnp.float32)


def _k3(h1_ref, st1_ref, g1_ref, be1_ref, a2bd_ref, b2_ref, out_ref, *, T, M):
    O1 = g1_ref.shape[1]
    Bnh = out_ref.shape[0] // 2
    st = st1_ref[...]
    sc1, sh1 = _affine_rows(st[:, :O1] + st[:, O1:], g1_ref[...], be1_ref[...], M)
    pack = lambda v: jnp.concatenate([v, v], axis=1)
    y1 = jnp.maximum(h1_ref[...].astype(jnp.float32) * pack(sc1) + pack(sh1), 0.0)
    s = jnp.dot(y1.astype(jnp.bfloat16), a2bd_ref[...],
                preferred_element_type=jnp.float32)
    ct = _combine_taps3(s[:, 0:3].astype(jnp.bfloat16), Bnh, T)
    cb = _combine_taps3(s[:, O1:O1 + 3].astype(jnp.bfloat16), Bnh, T)
    out_ref[...] = jnp.concatenate([ct, cb], axis=0) + b2_ref[...]


def kernel(x, W0, b0, g0, be0, W1, b1, g1, be1, W2, b2):
    N, T, H = x.shape
    O0 = W0.shape[0]
    O1 = W1.shape[0]
    M = N * T
    Bn = next((b for b in (400, 200, 80, 40, 16, 8) if N % b == 0), N)
    nb = N // Bn
    R = Bn * T

    acat0 = jnp.concatenate([jnp.transpose(W0[:, :, k]) for k in range(3)],
                            axis=0).astype(jnp.bfloat16)
    acat1 = jnp.concatenate([jnp.transpose(W1[:, :, k]) for k in range(3)],
                            axis=0).astype(jnp.bfloat16)
    a2bd = jnp.zeros((2 * O1, 2 * O1), jnp.float32)
    a2bd = a2bd.at[:O1, 0:3].set(W2[0]).at[O1:, O1:O1 + 3].set(W2[0])
    a2bd = a2bd.astype(jnp.bfloat16)
    b2row = jnp.broadcast_to(b2.reshape(1, 1), (1, T)).astype(jnp.float32)

    seq = pltpu.CompilerParams(dimension_semantics=("arbitrary",))
    wspec = lambda shape: pl.BlockSpec(shape, lambda i: (0, 0))

    h0, st0 = pl.pallas_call(
        functools.partial(_k1, T=T),
        grid=(nb,),
        in_specs=[
            pl.BlockSpec((R, H), lambda i: (i, 0)),
            wspec((3 * H, O0)),
            wspec((1, O0)),
        ],
        out_specs=(
            pl.BlockSpec((R, O0), lambda i: (i, 0)),
            wspec((16, O0)),
        ),
        out_shape=(
            jax.ShapeDtypeStruct((M, O0), jnp.bfloat16),
            jax.ShapeDtypeStruct((16, O0), jnp.float32),
        ),
        compiler_params=seq,
    )(x.reshape(M, H), acat0, b0.reshape(1, -1))

    h1, st1 = pl.pallas_call(
        functools.partial(_k2, T=T, M=M),
        grid=(nb,),
        in_specs=[
            pl.BlockSpec((R, O0), lambda i: (i, 0)),
            wspec((16, O0)), wspec((1, O0)), wspec((1, O0)),
            wspec((3 * O0, O1)),
            wspec((1, O1)),
        ],
        out_specs=(
            pl.BlockSpec((R // 2, 2 * O1), lambda i: (i, 0)),
            wspec((16, 2 * O1)),
        ),
        out_shape=(
            jax.ShapeDtypeStruct((M // 2, 2 * O1), jnp.bfloat16),
            jax.ShapeDtypeStruct((16, 2 * O1), jnp.float32),
        ),
        compiler_params=seq,
    )(h0, st0, g0.reshape(1, -1), be0.reshape(1, -1), acat1, b1.reshape(1, -1))

    out2d = pl.pallas_call(
        functools.partial(_k3, T=T, M=M),
        grid=(nb,),
        in_specs=[
            pl.BlockSpec((R // 2, 2 * O1), lambda i: (i, 0)),
            wspec((16, 2 * O1)), wspec((1, O1)), wspec((1, O1)),
            wspec((2 * O1, 2 * O1)),
            wspec((1, T)),
        ],
        out_specs=pl.BlockSpec((Bn, T), lambda i: (i, 0)),
        out_shape=jax.ShapeDtypeStruct((N, T), jnp.float32),
        compiler_params=seq,
    )(h1, st1, g1.reshape(1, -1), be1.reshape(1, -1), a2bd, b2row)

    return out2d[:, :, None]


# speedup vs baseline: 1.0231x; 1.0087x over previous
import functools

import jax
import jax.numpy as jnp
from jax.experimental import pallas as pl
from jax.experimental.pallas import tpu as pltpu

_EPS = 1e-5


def _conv3(y, acat, T):
    R, C = y.shape
    t = jax.lax.broadcasted_iota(jnp.int32, (R, 1), 0) % T
    z = jnp.zeros((1, C), jnp.bfloat16)
    zero = jnp.zeros((), jnp.bfloat16)
    yd = jnp.where(t == 0, zero, jnp.concatenate([z, y[:-1]], axis=0))
    yu = jnp.where(t == T - 1, zero, jnp.concatenate([y[1:], z], axis=0))
    ycat = jnp.concatenate([yd, y, yu], axis=1)
    return jnp.dot(ycat, acat, preferred_element_type=jnp.float32)


def _acc_stats(st_ref, h):
    C = h.shape[1]
    h3 = h.reshape(-1, 8, C)
    blk = jnp.concatenate([jnp.sum(h3, axis=0), jnp.sum(h3 * h3, axis=0)], axis=0)

    @pl.when(pl.program_id(0) == 0)
    def _():
        st_ref[...] = jnp.zeros_like(st_ref)

    st_ref[...] += blk


def _k1(x_ref, acat_ref, b0_ref, h0_ref, st_ref, *, T):
    xf = x_ref[...].astype(jnp.bfloat16)
    h0 = _conv3(xf, acat_ref[...], T) + b0_ref[...]
    h0_ref[...] = h0.astype(jnp.bfloat16)
    _acc_stats(st_ref, h0)


def _affine_rows(st, g, be, m):
    mean = jnp.sum(st[0:8], axis=0, keepdims=True) / m
    var = jnp.sum(st[8:16], axis=0, keepdims=True) / m - mean * mean
    inv = g * jax.lax.rsqrt(var + _EPS)
    return inv, be - mean * inv


def _k2(h0_ref, st0_ref, g0_ref, be0_ref, acat_ref, b1_ref, h1_ref, st_ref,
        *, T, M):
    sc0, sh0 = _affine_rows(st0_ref[...], g0_ref[...], be0_ref[...], M)
    y0 = jnp.maximum(h0_ref[...] * sc0.astype(jnp.bfloat16)
                     + sh0.astype(jnp.bfloat16), jnp.bfloat16(0))
    h1 = _conv3(y0, acat_ref[...], T) + b1_ref[...]
    half = h1.shape[0] // 2
    h1w = jnp.concatenate([h1[:half], h1[half:]], axis=1)
    h1_ref[...] = h1w.astype(jnp.bfloat16)
    _acc_stats(st_ref, h1w)


def _combine_taps3(s3, Bnh, T):
    v = s3.reshape(Bnh, T, 3)
    z = jnp.zeros((Bnh, 1, 1), jnp.bfloat16)
    c0 = jnp.concatenate([z, v[:, :-1, 0:1]], axis=1)
    c2 = jnp.concatenate([v[:, 1:, 2:3], z], axis=1)
    comb = c0 + v[:, :, 1:2] + c2
    return jnp.swapaxes(comb, 1, 2)[:, 0, :].astype(jnp.float32)


def _k3(h1_ref, st1_ref, g1_ref, be1_ref, a2bd_ref, b2_ref, out_ref, *, T, M):
    O1 = g1_ref.shape[1]
    Bnh = out_ref.shape[0] // 2
    st = st1_ref[...]
    sc1, sh1 = _affine_rows(st[:, :O1] + st[:, O1:], g1_ref[...], be1_ref[...], M)
    pack = lambda v: jnp.concatenate([v, v], axis=1)
    y1 = jnp.maximum(h1_ref[...].astype(jnp.float32) * pack(sc1) + pack(sh1), 0.0)
    s = jnp.dot(y1.astype(jnp.bfloat16), a2bd_ref[...],
                preferred_element_type=jnp.float32)
    ct = _combine_taps3(s[:, 0:3].astype(jnp.bfloat16), Bnh, T)
    cb = _combine_taps3(s[:, O1:O1 + 3].astype(jnp.bfloat16), Bnh, T)
    out_ref[...] = jnp.concatenate([ct, cb], axis=0) + b2_ref[...]


def kernel(x, W0, b0, g0, be0, W1, b1, g1, be1, W2, b2):
    N, T, H = x.shape
    O0 = W0.shape[0]
    O1 = W1.shape[0]
    M = N * T
    Bn = next((b for b in (400, 200, 80, 40, 16, 8) if N % b == 0), N)
    nb = N // Bn
    R = Bn * T

    acat0 = jnp.concatenate([jnp.transpose(W0[:, :, k]) for k in range(3)],
                            axis=0).astype(jnp.bfloat16)
    acat1 = jnp.concatenate([jnp.transpose(W1[:, :, k]) for k in range(3)],
                            axis=0).astype(jnp.bfloat16)
    a2bd = jnp.zeros((2 * O1, 2 * O1), jnp.float32)
    a2bd = a2bd.at[:O1, 0:3].set(W2[0]).at[O1:, O1:O1 + 3].set(W2[0])
    a2bd = a2bd.astype(jnp.bfloat16)
    b2row = jnp.broadcast_to(b2.reshape(1, 1), (1, T)).astype(jnp.float32)

    seq = pltpu.CompilerParams(dimension_semantics=("arbitrary",))
    wspec = lambda shape: pl.BlockSpec(shape, lambda i: (0, 0))

    h0, st0 = pl.pallas_call(
        functools.partial(_k1, T=T),
        grid=(nb,),
        in_specs=[
            pl.BlockSpec((R, H), lambda i: (i, 0)),
            wspec((3 * H, O0)),
            wspec((1, O0)),
        ],
        out_specs=(
            pl.BlockSpec((R, O0), lambda i: (i, 0)),
            wspec((16, O0)),
        ),
        out_shape=(
            jax.ShapeDtypeStruct((M, O0), jnp.bfloat16),
            jax.ShapeDtypeStruct((16, O0), jnp.float32),
        ),
        compiler_params=seq,
    )(x.reshape(M, H), acat0, b0.reshape(1, -1))

    h1, st1 = pl.pallas_call(
        functools.partial(_k2, T=T, M=M),
        grid=(nb,),
        in_specs=[
            pl.BlockSpec((R, O0), lambda i: (i, 0)),
            wspec((16, O0)), wspec((1, O0)), wspec((1, O0)),
            wspec((3 * O0, O1)),
            wspec((1, O1)),
        ],
        out_specs=(
            pl.BlockSpec((R // 2, 2 * O1), lambda i: (i, 0)),
            wspec((16, 2 * O1)),
        ),
        out_shape=(
            jax.ShapeDtypeStruct((M // 2, 2 * O1), jnp.bfloat16),
            jax.ShapeDtypeStruct((16, 2 * O1), jnp.float32),
        ),
        compiler_params=seq,
    )(h0, st0, g0.reshape(1, -1), be0.reshape(1, -1), acat1, b1.reshape(1, -1))

    out2d = pl.pallas_call(
        functools.partial(_k3, T=T, M=M),
        grid=(nb,),
        in_specs=[
            pl.BlockSpec((R // 2, 2 * O1), lambda i: (i, 0)),
            wspec((16, 2 * O1)), wspec((1, O1)), wspec((1, O1)),
            wspec((2 * O1, 2 * O1)),
            wspec((1, T)),
        ],
        out_specs=pl.BlockSpec((Bn, T), lambda i: (i, 0)),
        out_shape=jax.ShapeDtypeStruct((N, T), jnp.float32),
        compiler_params=seq,
    )(h1, st1, g1.reshape(1, -1), be1.reshape(1, -1), a2bd, b2row)

    return out2d[:, :, None]


# speedup vs baseline: 1.0456x; 1.0219x over previous
import functools

import jax
import jax.numpy as jnp
from jax.experimental import pallas as pl
from jax.experimental.pallas import tpu as pltpu

_EPS = 1e-5


def _conv3(y, acat, T):
    R, C = y.shape
    v = y.reshape(R // T, T, C)
    z = jnp.zeros((R // T, 1, C), jnp.bfloat16)
    yd = jnp.concatenate([z, v[:, :-1]], axis=1).reshape(R, C)
    yu = jnp.concatenate([v[:, 1:], z], axis=1).reshape(R, C)
    ycat = jnp.concatenate([yd, y, yu], axis=1)
    return jnp.dot(ycat, acat, preferred_element_type=jnp.float32)


def _acc_stats(st_ref, h):
    C = h.shape[1]
    h3 = h.reshape(-1, 8, C)
    blk = jnp.concatenate([jnp.sum(h3, axis=0), jnp.sum(h3 * h3, axis=0)], axis=0)

    @pl.when(pl.program_id(0) == 0)
    def _():
        st_ref[...] = jnp.zeros_like(st_ref)

    st_ref[...] += blk


def _k1(x_ref, acat_ref, b0_ref, h0_ref, st_ref, *, T):
    xf = x_ref[...].astype(jnp.bfloat16)
    h0 = _conv3(xf, acat_ref[...], T) + b0_ref[...]
    h0_ref[...] = h0.astype(jnp.bfloat16)
    _acc_stats(st_ref, h0)


def _affine_rows(st, g, be, m):
    mean = jnp.sum(st[0:8], axis=0, keepdims=True) / m
    var = jnp.sum(st[8:16], axis=0, keepdims=True) / m - mean * mean
    inv = g * jax.lax.rsqrt(var + _EPS)
    return inv, be - mean * inv


def _k2(h0_ref, st0_ref, g0_ref, be0_ref, acat_ref, b1_ref, h1_ref, st_ref,
        *, T, M):
    sc0, sh0 = _affine_rows(st0_ref[...], g0_ref[...], be0_ref[...], M)
    y0 = jnp.maximum(h0_ref[...] * sc0.astype(jnp.bfloat16)
                     + sh0.astype(jnp.bfloat16), jnp.bfloat16(0))
    h1 = _conv3(y0, acat_ref[...], T) + b1_ref[...]
    half = h1.shape[0] // 2
    h1w = jnp.concatenate([h1[:half], h1[half:]], axis=1)
    h1_ref[...] = h1w.astype(jnp.bfloat16)
    _acc_stats(st_ref, h1w)


def _combine_taps3(s3, Bnh, T):
    v = s3.reshape(Bnh, T, 3)
    z = jnp.zeros((Bnh, 1, 1), jnp.bfloat16)
    c0 = jnp.concatenate([z, v[:, :-1, 0:1]], axis=1)
    c2 = jnp.concatenate([v[:, 1:, 2:3], z], axis=1)
    comb = c0 + v[:, :, 1:2] + c2
    return jnp.swapaxes(comb, 1, 2)[:, 0, :].astype(jnp.float32)


def _k3(h1_ref, st1_ref, g1_ref, be1_ref, a2bd_ref, b2_ref, out_ref, *, T, M):
    O1 = g1_ref.shape[1]
    Bnh = out_ref.shape[0] // 2
    st = st1_ref[...]
    sc1, sh1 = _affine_rows(st[:, :O1] + st[:, O1:], g1_ref[...], be1_ref[...], M)
    pack = lambda v: jnp.concatenate([v, v], axis=1)
    y1 = jnp.maximum(h1_ref[...].astype(jnp.float32) * pack(sc1) + pack(sh1), 0.0)
    s = jnp.dot(y1.astype(jnp.bfloat16), a2bd_ref[...],
                preferred_element_type=jnp.float32)
    ct = _combine_taps3(s[:, 0:3].astype(jnp.bfloat16), Bnh, T)
    cb = _combine_taps3(s[:, O1:O1 + 3].astype(jnp.bfloat16), Bnh, T)
    out_ref[...] = jnp.concatenate([ct, cb], axis=0) + b2_ref[...]


def kernel(x, W0, b0, g0, be0, W1, b1, g1, be1, W2, b2):
    N, T, H = x.shape
    O0 = W0.shape[0]
    O1 = W1.shape[0]
    M = N * T
    Bn = next((b for b in (400, 200, 80, 40, 16, 8) if N % b == 0), N)
    nb = N // Bn
    R = Bn * T

    acat0 = jnp.concatenate([jnp.transpose(W0[:, :, k]) for k in range(3)],
                            axis=0).astype(jnp.bfloat16)
    acat1 = jnp.concatenate([jnp.transpose(W1[:, :, k]) for k in range(3)],
                            axis=0).astype(jnp.bfloat16)
    a2bd = jnp.zeros((2 * O1, 2 * O1), jnp.float32)
    a2bd = a2bd.at[:O1, 0:3].set(W2[0]).at[O1:, O1:O1 + 3].set(W2[0])
    a2bd = a2bd.astype(jnp.bfloat16)
    b2row = jnp.broadcast_to(b2.reshape(1, 1), (1, T)).astype(jnp.float32)

    seq = pltpu.CompilerParams(dimension_semantics=("arbitrary",))
    wspec = lambda shape: pl.BlockSpec(shape, lambda i: (0, 0))

    h0, st0 = pl.pallas_call(
        functools.partial(_k1, T=T),
        grid=(nb,),
        in_specs=[
            pl.BlockSpec((R, H), lambda i: (i, 0)),
            wspec((3 * H, O0)),
            wspec((1, O0)),
        ],
        out_specs=(
            pl.BlockSpec((R, O0), lambda i: (i, 0)),
            wspec((16, O0)),
        ),
        out_shape=(
            jax.ShapeDtypeStruct((M, O0), jnp.bfloat16),
            jax.ShapeDtypeStruct((16, O0), jnp.float32),
        ),
        compiler_params=seq,
    )(x.reshape(M, H), acat0, b0.reshape(1, -1))

    h1, st1 = pl.pallas_call(
        functools.partial(_k2, T=T, M=M),
        grid=(nb,),
        in_specs=[
            pl.BlockSpec((R, O0), lambda i: (i, 0)),
            wspec((16, O0)), wspec((1, O0)), wspec((1, O0)),
            wspec((3 * O0, O1)),
            wspec((1, O1)),
        ],
        out_specs=(
            pl.BlockSpec((R // 2, 2 * O1), lambda i: (i, 0)),
            wspec((16, 2 * O1)),
        ),
        out_shape=(
            jax.ShapeDtypeStruct((M // 2, 2 * O1), jnp.bfloat16),
            jax.ShapeDtypeStruct((16, 2 * O1), jnp.float32),
        ),
        compiler_params=seq,
    )(h0, st0, g0.reshape(1, -1), be0.reshape(1, -1), acat1, b1.reshape(1, -1))

    out2d = pl.pallas_call(
        functools.partial(_k3, T=T, M=M),
        grid=(nb,),
        in_specs=[
            pl.BlockSpec((R // 2, 2 * O1), lambda i: (i, 0)),
            wspec((16, 2 * O1)), wspec((1, O1)), wspec((1, O1)),
            wspec((2 * O1, 2 * O1)),
            wspec((1, T)),
        ],
        out_specs=pl.BlockSpec((Bn, T), lambda i: (i, 0)),
        out_shape=jax.ShapeDtypeStruct((N, T), jnp.float32),
        compiler_params=seq,
    )(h1, st1, g1.reshape(1, -1), be1.reshape(1, -1), a2bd, b2row)

    return out2d[:, :, None]


# speedup vs baseline: 1.0929x; 1.0453x over previous
import functools

import jax
import jax.numpy as jnp
from jax.experimental import pallas as pl
from jax.experimental.pallas import tpu as pltpu

_EPS = 1e-5


def _cat3(y, T):
    R, C = y.shape
    v = y.reshape(R // T, T, C)
    z = jnp.zeros((R // T, 1, C), jnp.bfloat16)
    yd = jnp.concatenate([z, v[:, :-1]], axis=1).reshape(R, C)
    yu = jnp.concatenate([v[:, 1:], z], axis=1).reshape(R, C)
    return jnp.concatenate([yd, y, yu], axis=1)


def _conv3(y, acat, T):
    return jnp.dot(_cat3(y, T), acat, preferred_element_type=jnp.float32)


def _acc_stats(st_ref, h):
    C = h.shape[1]
    h3 = h.reshape(-1, 8, C)
    blk = jnp.concatenate([jnp.sum(h3, axis=0), jnp.sum(h3 * h3, axis=0)], axis=0)

    @pl.when(pl.program_id(0) == 0)
    def _():
        st_ref[...] = jnp.zeros_like(st_ref)

    st_ref[...] += blk


def _k1(x_ref, acat_ref, b0_ref, h0_ref, st_ref, *, T):
    xf = x_ref[...].astype(jnp.bfloat16)
    h0 = _conv3(xf, acat_ref[...], T) + b0_ref[...]
    h0_ref[...] = h0.astype(jnp.bfloat16)
    _acc_stats(st_ref, h0)


def _affine_rows(st, g, be, m):
    mean = jnp.sum(st[0:8], axis=0, keepdims=True) / m
    var = jnp.sum(st[8:16], axis=0, keepdims=True) / m - mean * mean
    inv = g * jax.lax.rsqrt(var + _EPS)
    return inv, be - mean * inv


def _k2(h0_ref, st0_ref, g0_ref, be0_ref, acat_ref, b1_ref, h1_ref, st_ref,
        *, T, M):
    sc0, sh0 = _affine_rows(st0_ref[...], g0_ref[...], be0_ref[...], M)
    y0 = jnp.maximum(h0_ref[...] * sc0.astype(jnp.bfloat16)
                     + sh0.astype(jnp.bfloat16), jnp.bfloat16(0))
    h1 = _conv3(y0, acat_ref[...], T) + b1_ref[...]
    half = h1.shape[0] // 2
    h1w = jnp.concatenate([h1[:half], h1[half:]], axis=1)
    h1_ref[...] = h1w.astype(jnp.bfloat16)
    _acc_stats(st_ref, h1w)


def _col_rows(col, Bnh, T):
    return jnp.swapaxes(col.reshape(Bnh, T, 1), 1, 2)[:, 0, :].astype(jnp.float32)


def _k3(h1_ref, st1_ref, g1_ref, be1_ref, a3_ref, b2_ref, out_ref, *, T, M):
    O1 = g1_ref.shape[1]
    Bnh = out_ref.shape[0] // 2
    st = st1_ref[...]
    sc1, sh1 = _affine_rows(st[:, :O1] + st[:, O1:], g1_ref[...], be1_ref[...], M)
    pack = lambda v: jnp.concatenate([v, v], axis=1)
    y1 = jnp.maximum(h1_ref[...].astype(jnp.float32) * pack(sc1) + pack(sh1), 0.0)
    s = jnp.dot(_cat3(y1.astype(jnp.bfloat16), T), a3_ref[...],
                preferred_element_type=jnp.float32).astype(jnp.bfloat16)
    ct = _col_rows(s[:, 0:1], Bnh, T)
    cb = _col_rows(s[:, 1:2], Bnh, T)
    out_ref[...] = jnp.concatenate([ct, cb], axis=0) + b2_ref[...]


def kernel(x, W0, b0, g0, be0, W1, b1, g1, be1, W2, b2):
    N, T, H = x.shape
    O0 = W0.shape[0]
    O1 = W1.shape[0]
    M = N * T
    Bn = next((b for b in (400, 200, 80, 40, 16, 8) if N % b == 0), N)
    nb = N // Bn
    R = Bn * T

    acat0 = jnp.concatenate([jnp.transpose(W0[:, :, k]) for k in range(3)],
                            axis=0).astype(jnp.bfloat16)
    acat1 = jnp.concatenate([jnp.transpose(W1[:, :, k]) for k in range(3)],
                            axis=0).astype(jnp.bfloat16)
    a3 = jnp.zeros((3 * 2 * O1, 8), jnp.float32)
    for k in range(3):
        a3 = a3.at[2 * O1 * k:2 * O1 * k + O1, 0].set(W2[0, :, k])
        a3 = a3.at[2 * O1 * k + O1:2 * O1 * (k + 1), 1].set(W2[0, :, k])
    a3 = a3.astype(jnp.bfloat16)
    b2row = jnp.broadcast_to(b2.reshape(1, 1), (1, T)).astype(jnp.float32)

    seq = pltpu.CompilerParams(dimension_semantics=("arbitrary",))
    wspec = lambda shape: pl.BlockSpec(shape, lambda i: (0, 0))

    h0, st0 = pl.pallas_call(
        functools.partial(_k1, T=T),
        grid=(nb,),
        in_specs=[
            pl.BlockSpec((R, H), lambda i: (i, 0)),
            wspec((3 * H, O0)),
            wspec((1, O0)),
        ],
        out_specs=(
            pl.BlockSpec((R, O0), lambda i: (i, 0)),
            wspec((16, O0)),
        ),
        out_shape=(
            jax.ShapeDtypeStruct((M, O0), jnp.bfloat16),
            jax.ShapeDtypeStruct((16, O0), jnp.float32),
        ),
        compiler_params=seq,
    )(x.reshape(M, H), acat0, b0.reshape(1, -1))

    h1, st1 = pl.pallas_call(
        functools.partial(_k2, T=T, M=M),
        grid=(nb,),
        in_specs=[
            pl.BlockSpec((R, O0), lambda i: (i, 0)),
            wspec((16, O0)), wspec((1, O0)), wspec((1, O0)),
            wspec((3 * O0, O1)),
            wspec((1, O1)),
        ],
        out_specs=(
            pl.BlockSpec((R // 2, 2 * O1), lambda i: (i, 0)),
            wspec((16, 2 * O1)),
        ),
        out_shape=(
            jax.ShapeDtypeStruct((M // 2, 2 * O1), jnp.bfloat16),
            jax.ShapeDtypeStruct((16, 2 * O1), jnp.float32),
        ),
        compiler_params=seq,
    )(h0, st0, g0.reshape(1, -1), be0.reshape(1, -1), acat1, b1.reshape(1, -1))

    out2d = pl.pallas_call(
        functools.partial(_k3, T=T, M=M),
        grid=(nb,),
        in_specs=[
            pl.BlockSpec((R // 2, 2 * O1), lambda i: (i, 0)),
            wspec((16, 2 * O1)), wspec((1, O1)), wspec((1, O1)),
            wspec((3 * 2 * O1, 8)),
            wspec((1, T)),
        ],
        out_specs=pl.BlockSpec((Bn, T), lambda i: (i, 0)),
        out_shape=jax.ShapeDtypeStruct((N, T), jnp.float32),
        compiler_params=seq,
    )(h1, st1, g1.reshape(1, -1), be1.reshape(1, -1), a3, b2row)

    return out2d[:, :, None]
